# baseline (speedup 1.0000x reference)
import functools

import jax
import jax.numpy as jnp
from jax import lax
from jax.experimental import pallas as pl
from jax.experimental.pallas import tpu as pltpu
from jax.experimental.pallas import tpu_sc as plsc

_NC = 2
_NS = 16
_NW = _NC * _NS
_CH = 128
_R = 8


def _ceil_to(v, m):
    return -(-v // m) * m


@functools.lru_cache(None)
def _sc_degree(n_pad, rows_per_tile):
    slice_rows = n_pad // _NS
    n_super = rows_per_tile // _R
    mesh = plsc.VectorSubcoreMesh(core_axis_name="c", subcore_axis_name="s")

    def body(dst_hbm, ones_hbm, z_hbm, out_hbm, dst_v, ones_v, acc, sem):
        c = lax.axis_index("c")
        s = lax.axis_index("s")
        pltpu.sync_copy(z_hbm, acc.at[pl.ds(s * slice_rows, slice_rows)])
        pltpu.sync_copy(ones_hbm, ones_v)
        plsc.subcore_barrier()
        base = (c * _NS + s) * rows_per_tile

        def step(i, carry):
            b = lax.rem(i, 2)
            row0 = base + i * _R

            @pl.when(i >= 2)
            def _():
                for j in range(_R):
                    pltpu.make_async_copy(
                        ones_v, acc.at[dst_v.at[b, j]], sem).wait()

            pltpu.sync_copy(dst_hbm.at[pl.ds(row0, _R)], dst_v.at[b])
            for j in range(_R):
                pltpu.async_copy(ones_v, acc.at[dst_v.at[b, j]], sem,
                                 add=True)
            return carry

        lax.fori_loop(0, n_super, step, 0)
        for slot in range(2):
            for j in range(_R):
                pltpu.make_async_copy(
                    ones_v, acc.at[dst_v.at[slot, j]], sem).wait()
        plsc.subcore_barrier()
        pltpu.sync_copy(acc.at[pl.ds(s * slice_rows, slice_rows)],
                        out_hbm.at[c, pl.ds(s * slice_rows, slice_rows)])

    return pl.kernel(
        body,
        out_type=jax.ShapeDtypeStruct((_NC, n_pad, 8), jnp.float32),
        mesh=mesh,
        compiler_params=pltpu.CompilerParams(use_tc_tiling_on_sc=False),
        scratch_types=[
            pltpu.VMEM((2, _R, _CH), jnp.int32),
            pltpu.VMEM((_CH, 8), jnp.float32),
            pltpu.VMEM_SHARED((n_pad, 8), jnp.float32),
            pltpu.SemaphoreType.DMA,
        ],
    )


@functools.lru_cache(None)
def _sc_agg(d, n_pad, rows_per_tile, edge_split, r=_R):
    slice_rows = n_pad // _NS
    n_super = rows_per_tile // r
    mesh = plsc.VectorSubcoreMesh(core_axis_name="c", subcore_axis_name="s")

    def body(ei_hbm, g_hbm, z_hbm, out_hbm,
             idx_v, rows_v, acc, gsem, ssem):
        c = lax.axis_index("c")
        s = lax.axis_index("s")
        pltpu.sync_copy(z_hbm, acc.at[pl.ds(s * slice_rows, slice_rows)])
        plsc.subcore_barrier()
        if edge_split:
            base = (c * _NS + s) * rows_per_tile
        else:
            base = s * rows_per_tile

        def step(i, carry):
            b = lax.rem(i, 2)
            row0 = base + i * r

            @pl.when(i >= 2)
            def _():
                for j in range(r):
                    pltpu.make_async_copy(
                        rows_v.at[b, j], acc.at[idx_v.at[b, j, 1]],
                        ssem).wait()

            if edge_split:
                pltpu.sync_copy(ei_hbm.at[pl.ds(row0, r)], idx_v.at[b])
            else:
                pltpu.sync_copy(ei_hbm.at[c, pl.ds(row0, r)], idx_v.at[b])
            gds = [pltpu.async_copy(g_hbm.at[idx_v.at[b, j, 0]],
                                    rows_v.at[b, j], gsem)
                   for j in range(r)]
            for dd in gds:
                dd.wait()
            for j in range(r):
                pltpu.async_copy(rows_v.at[b, j], acc.at[idx_v.at[b, j, 1]],
                                 ssem, add=True)
            return carry

        lax.fori_loop(0, n_super, step, 0)
        for slot in range(2):
            for j in range(r):
                pltpu.make_async_copy(
                    rows_v.at[slot, j], acc.at[idx_v.at[slot, j, 1]],
                    ssem).wait()
        plsc.subcore_barrier()
        pltpu.sync_copy(acc.at[pl.ds(s * slice_rows, slice_rows)],
                        out_hbm.at[c, pl.ds(s * slice_rows, slice_rows)])

    return pl.kernel(
        body,
        out_type=jax.ShapeDtypeStruct((_NC, n_pad, d), jnp.float32),
        mesh=mesh,
        compiler_params=pltpu.CompilerParams(use_tc_tiling_on_sc=False),
        scratch_types=[
            pltpu.VMEM((2, r, 2, _CH), jnp.int32),
            pltpu.VMEM((2, r, _CH, d), jnp.float32),
            pltpu.VMEM_SHARED((n_pad, d), jnp.float32),
            pltpu.SemaphoreType.DMA,
            pltpu.SemaphoreType.DMA,
        ],
    )


def _t0_body(degp_ref, x_ref, dinv_ref, g0_ref):
    deg = degp_ref[0, :, 0:1] + degp_ref[1, :, 0:1] + 1.0
    dinv = lax.rsqrt(deg)
    dinv_ref[...] = dinv
    g0 = x_ref[...] * dinv
    g0_ref[...] = jnp.concatenate(
        [g0, jnp.zeros((g0.shape[0], 6), jnp.float32)], axis=1)


@functools.lru_cache(None)
def _t0(n, n_pad, bn, interpret=False):
    return pl.pallas_call(
        _t0_body,
        grid=(n // bn,),
        compiler_params=pltpu.CompilerParams(allow_input_fusion=[True] * 2),
        in_specs=[pl.BlockSpec((2, bn, 8), lambda i: (0, i, 0)),
                  pl.BlockSpec((bn, 2), lambda i: (i, 0))],
        out_specs=[pl.BlockSpec((bn, 1), lambda i: (i, 0)),
                   pl.BlockSpec((bn, 8), lambda i: (i, 0))],
        out_shape=[jax.ShapeDtypeStruct((n, 1), jnp.float32),
                   jax.ShapeDtypeStruct((n, 8), jnp.float32)],
        interpret=interpret,
    )


def _layer_body_single(p_ref, g_ref, dinv_ref, w_ref, b_ref, out_ref):
    dinv = dinv_ref[...]
    agg = (p_ref[0] + p_ref[1] + g_ref[...]) * dinv
    h = jnp.dot(agg, w_ref[...], preferred_element_type=jnp.float32)
    h = jnp.maximum(h + b_ref[...], 0.0)
    out_ref[...] = h * dinv


def _layer_body_halves(p_ref, g_ref, dinv_ref, w_ref, b_ref, out_ref):
    dinv = dinv_ref[...]
    agg = (p_ref[0] + p_ref[1] + g_ref[...]) * dinv
    h = jnp.dot(agg, w_ref[...], preferred_element_type=jnp.float32)
    h = jnp.maximum(h + b_ref[...], 0.0)
    g = h * dinv
    half = g.shape[1] // 2
    out_ref[0] = g[:, :half]
    out_ref[1] = g[:, half:]


@functools.lru_cache(None)
def _t_layer(n, n_pad, bn, din, dout, halves, interpret=False):
    in_specs = [pl.BlockSpec((2, bn, din), lambda i: (0, i, 0)),
                pl.BlockSpec((bn, din), lambda i: (i, 0)),
                pl.BlockSpec((bn, 1), lambda i: (i, 0)),
                pl.BlockSpec((din, dout), lambda i: (0, 0)),
                pl.BlockSpec((1, dout), lambda i: (0, 0))]
    if halves:
        body = _layer_body_halves
        out_specs = pl.BlockSpec((2, bn, dout // 2), lambda i: (0, i, 0))
        out_shape = jax.ShapeDtypeStruct((2, n, dout // 2), jnp.float32)
    else:
        body = _layer_body_single
        out_specs = pl.BlockSpec((bn, dout), lambda i: (i, 0))
        out_shape = jax.ShapeDtypeStruct((n, dout), jnp.float32)
    return pl.pallas_call(
        body,
        grid=(n // bn,),
        compiler_params=pltpu.CompilerParams(allow_input_fusion=[True] * 5),
        in_specs=in_specs,
        out_specs=out_specs,
        out_shape=out_shape,
        interpret=interpret,
    )


def _t3_body(p_ref, g2_ref, dinv_ref, w_ref, b_ref, lg_ref, gn_ref,
             out_ref, *, temp):
    i = pl.program_id(0)
    dinv = dinv_ref[...]
    agg_a = (p_ref[0] + g2_ref[0]) * dinv
    agg_b = (p_ref[1] + g2_ref[1]) * dinv
    half = w_ref.shape[0] // 2
    h = (jnp.dot(agg_a, w_ref[:half], preferred_element_type=jnp.float32)
         + jnp.dot(agg_b, w_ref[half:], preferred_element_type=jnp.float32))
    h = jnp.maximum(h + b_ref[...], 0.0)
    t = (lg_ref[...] + gn_ref[...]) * (1.0 / temp)
    t = t - jnp.max(t, axis=1, keepdims=True)
    e = jnp.exp(t)
    y = e / jnp.sum(e, axis=1, keepdims=True)
    contrib = lax.dot_general(y, h, (((0,), (0,)), ((), ())),
                              preferred_element_type=jnp.float32)

    @pl.when(i == 0)
    def _():
        out_ref[...] = contrib

    @pl.when(i > 0)
    def _():
        out_ref[...] += contrib


@functools.lru_cache(None)
def _t3(n, n_pad, bn, k, dout, temp, interpret=False):
    return pl.pallas_call(
        functools.partial(_t3_body, temp=temp),
        grid=(n // bn,),
        compiler_params=pltpu.CompilerParams(allow_input_fusion=[True] * 8),
        in_specs=[pl.BlockSpec((2, bn, 16), lambda i: (0, i, 0)),
                  pl.BlockSpec((2, bn, 16), lambda i: (0, i, 0)),
                  pl.BlockSpec((bn, 1), lambda i: (i, 0)),
                  pl.BlockSpec((32, dout), lambda i: (0, 0)),
                  pl.BlockSpec((1, dout), lambda i: (0, 0)),
                  pl.BlockSpec((bn, k), lambda i: (i, 0)),
                  pl.BlockSpec((bn, k), lambda i: (i, 0))],
        out_specs=pl.BlockSpec((k, dout), lambda i: (0, 0)),
        out_shape=jax.ShapeDtypeStruct((k, dout), jnp.float32),
        interpret=interpret,
    )


def kernel(x, edge_index, logits, gumbel_noise, W1, b1, W2, b2, W3, b3):
    n = x.shape[0]
    e = edge_index.shape[1]
    k = logits.shape[1]
    bn = 2000
    temp = 0.5

    e_pad = _ceil_to(e, _NW * _R * _CH)
    n_pad = _ceil_to(n + 1, _NS * 8 * 2)
    trash = n

    src = edge_index[0]
    dst = edge_index[1]
    pad_e = e_pad - e
    src_p = jnp.concatenate([src, jnp.zeros((pad_e,), jnp.int32)])
    dst_p = jnp.concatenate([dst, jnp.full((pad_e,), trash, jnp.int32)])
    rows_idx = e_pad // _CH
    src2d = src_p.reshape(rows_idx, _CH)
    dst2d = dst_p.reshape(rows_idx, _CH)
    ei2 = jnp.stack([src2d, dst2d], axis=1)
    ei3 = jnp.stack([jnp.stack([src2d, dst2d], axis=1),
                     jnp.stack([src2d + n, dst2d], axis=1)])
    rpt_split = rows_idx // _NW
    rpt_all = rows_idx // _NS

    slice_rows = n_pad // _NS
    zeros8 = jnp.zeros((slice_rows, 8), jnp.float32)
    zeros16 = jnp.zeros((slice_rows, 16), jnp.float32)
    ones128 = jnp.ones((_CH, 8), jnp.float32)

    degp = _sc_degree(n_pad, rpt_split)(dst2d, ones128, zeros8)
    dinv, g0 = _t0(n, n_pad, bn)(degp, x)
    p1 = _sc_agg(8, n_pad, rpt_split, True)(ei2, g0, zeros8)
    w1p = jnp.concatenate([W1, jnp.zeros((6, W1.shape[1]), W1.dtype)], axis=0)
    g1 = _t_layer(n, n_pad, bn, 8, 16, False)(
        p1, g0, dinv, w1p, b1.reshape(1, -1))
    p2 = _sc_agg(16, n_pad, rpt_split, True, 4)(ei2, g1, zeros16)
    g2 = _t_layer(n, n_pad, bn, 16, 32, True)(
        p2, g1, dinv, W2, b2.reshape(1, -1))
    g2flat = g2.reshape(2 * n, 16)
    p3 = _sc_agg(16, n_pad, rpt_all, False, 4)(ei3, g2flat, zeros16)
    out = _t3(n, n_pad, bn, k, W3.shape[1], temp)(
        p3, g2, dinv, W3, b3.reshape(1, -1),
        logits, gumbel_noise)
    return out.reshape(1, -1)


# speedup vs baseline: 1.0912x; 1.0912x over previous
import functools

import jax
import jax.numpy as jnp
from jax import lax
from jax.experimental import pallas as pl
from jax.experimental.pallas import tpu as pltpu
from jax.experimental.pallas import tpu_sc as plsc

_NC = 2
_NS = 16
_NW = _NC * _NS
_CH = 128
_R = 8


def _ceil_to(v, m):
    return -(-v // m) * m


@functools.lru_cache(None)
def _sc_degree(n_pad, rows_per_tile):
    slice_rows = n_pad // _NS
    n_super = rows_per_tile // _R
    mesh = plsc.VectorSubcoreMesh(core_axis_name="c", subcore_axis_name="s")

    def body(dst_hbm, ones_hbm, z_hbm, out_hbm, dst_v, ones_v, acc, sem):
        c = lax.axis_index("c")
        s = lax.axis_index("s")
        pltpu.sync_copy(z_hbm, acc.at[pl.ds(s * slice_rows, slice_rows)])
        pltpu.sync_copy(ones_hbm, ones_v)
        plsc.subcore_barrier()
        base = (c * _NS + s) * rows_per_tile

        def step(i, carry):
            b = lax.rem(i, 2)
            row0 = base + i * _R

            @pl.when(i >= 2)
            def _():
                for j in range(_R):
                    pltpu.make_async_copy(
                        ones_v, acc.at[dst_v.at[b, j]], sem).wait()

            pltpu.sync_copy(dst_hbm.at[pl.ds(row0, _R)], dst_v.at[b])
            for j in range(_R):
                pltpu.async_copy(ones_v, acc.at[dst_v.at[b, j]], sem,
                                 add=True)
            return carry

        lax.fori_loop(0, n_super, step, 0)
        for slot in range(2):
            for j in range(_R):
                pltpu.make_async_copy(
                    ones_v, acc.at[dst_v.at[slot, j]], sem).wait()
        plsc.subcore_barrier()
        pltpu.sync_copy(acc.at[pl.ds(s * slice_rows, slice_rows)],
                        out_hbm.at[c, pl.ds(s * slice_rows, slice_rows)])

    return pl.kernel(
        body,
        out_type=jax.ShapeDtypeStruct((_NC, n_pad, 8), jnp.float32),
        mesh=mesh,
        compiler_params=pltpu.CompilerParams(use_tc_tiling_on_sc=False),
        scratch_types=[
            pltpu.VMEM((2, _R, _CH), jnp.int32),
            pltpu.VMEM((_CH, 8), jnp.float32),
            pltpu.VMEM_SHARED((n_pad, 8), jnp.float32),
            pltpu.SemaphoreType.DMA,
        ],
    )


@functools.lru_cache(None)
def _sc_agg(d, n_pad, rows_per_tile, edge_split, r=_R):
    slice_rows = n_pad // _NS
    n_super = rows_per_tile // r
    mesh = plsc.VectorSubcoreMesh(core_axis_name="c", subcore_axis_name="s")

    def body(ei_hbm, g_hbm, z_hbm, out_hbm,
             idx_v, rows_v, acc, gsem, ssem):
        c = lax.axis_index("c")
        s = lax.axis_index("s")
        pltpu.sync_copy(z_hbm, acc.at[pl.ds(s * slice_rows, slice_rows)])
        plsc.subcore_barrier()
        if edge_split:
            base = (c * _NS + s) * rows_per_tile
        else:
            base = s * rows_per_tile

        def step(i, carry):
            b = lax.rem(i, 2)
            row0 = base + i * r

            @pl.when(i >= 2)
            def _():
                for j in range(r):
                    pltpu.make_async_copy(
                        rows_v.at[b, j], acc.at[idx_v.at[b, j, 1]],
                        ssem).wait()

            if edge_split:
                pltpu.sync_copy(ei_hbm.at[pl.ds(row0, r)], idx_v.at[b])
            else:
                pltpu.sync_copy(ei_hbm.at[c, pl.ds(row0, r)], idx_v.at[b])
            gds = [pltpu.async_copy(g_hbm.at[idx_v.at[b, j, 0]],
                                    rows_v.at[b, j], gsem)
                   for j in range(r)]
            for dd in gds:
                dd.wait()
            for j in range(r):
                pltpu.async_copy(rows_v.at[b, j], acc.at[idx_v.at[b, j, 1]],
                                 ssem, add=True)
            return carry

        lax.fori_loop(0, n_super, step, 0)
        for slot in range(2):
            for j in range(r):
                pltpu.make_async_copy(
                    rows_v.at[slot, j], acc.at[idx_v.at[slot, j, 1]],
                    ssem).wait()
        plsc.subcore_barrier()
        pltpu.sync_copy(acc.at[pl.ds(s * slice_rows, slice_rows)],
                        out_hbm.at[c, pl.ds(s * slice_rows, slice_rows)])

    return pl.kernel(
        body,
        out_type=jax.ShapeDtypeStruct((_NC, n_pad, d), jnp.float32),
        mesh=mesh,
        compiler_params=pltpu.CompilerParams(use_tc_tiling_on_sc=False),
        scratch_types=[
            pltpu.VMEM((2, r, 2, _CH), jnp.int32),
            pltpu.VMEM((2, r, _CH, d), jnp.float32),
            pltpu.VMEM_SHARED((n_pad, d), jnp.float32),
            pltpu.SemaphoreType.DMA,
            pltpu.SemaphoreType.DMA,
        ],
    )


def _t0_body(degp_ref, x_ref, dinv_ref, g0_ref):
    deg = degp_ref[0, :, 0:1] + degp_ref[1, :, 0:1] + 1.0
    dinv = lax.rsqrt(deg)
    dinv_ref[...] = dinv
    g0 = x_ref[...] * dinv
    g0_ref[...] = jnp.concatenate(
        [g0, jnp.zeros((g0.shape[0], 6), jnp.float32)], axis=1)


@functools.lru_cache(None)
def _t0(n, n_pad, bn, interpret=False):
    return pl.pallas_call(
        _t0_body,
        grid=(n // bn,),
        in_specs=[pl.BlockSpec((2, bn, 8), lambda i: (0, i, 0)),
                  pl.BlockSpec((bn, 2), lambda i: (i, 0))],
        out_specs=[pl.BlockSpec((bn, 1), lambda i: (i, 0)),
                   pl.BlockSpec((bn, 8), lambda i: (i, 0))],
        out_shape=[jax.ShapeDtypeStruct((n, 1), jnp.float32),
                   jax.ShapeDtypeStruct((n, 8), jnp.float32)],
        interpret=interpret,
    )


def _layer_body_single(p_ref, g_ref, dinv_ref, w_ref, b_ref, out_ref):
    dinv = dinv_ref[...]
    agg = (p_ref[0] + p_ref[1] + g_ref[...]) * dinv
    h = jnp.dot(agg, w_ref[...], preferred_element_type=jnp.float32)
    h = jnp.maximum(h + b_ref[...], 0.0)
    out_ref[...] = h * dinv


def _layer_body_halves(p_ref, g_ref, dinv_ref, w_ref, b_ref, out_ref):
    dinv = dinv_ref[...]
    agg = (p_ref[0] + p_ref[1] + g_ref[...]) * dinv
    h = jnp.dot(agg, w_ref[...], preferred_element_type=jnp.float32)
    h = jnp.maximum(h + b_ref[...], 0.0)
    g = h * dinv
    half = g.shape[1] // 2
    out_ref[0] = g[:, :half]
    out_ref[1] = g[:, half:]


@functools.lru_cache(None)
def _t_layer(n, n_pad, bn, din, dout, halves, interpret=False):
    in_specs = [pl.BlockSpec((2, bn, din), lambda i: (0, i, 0)),
                pl.BlockSpec((bn, din), lambda i: (i, 0)),
                pl.BlockSpec((bn, 1), lambda i: (i, 0)),
                pl.BlockSpec((din, dout), lambda i: (0, 0)),
                pl.BlockSpec((1, dout), lambda i: (0, 0))]
    if halves:
        body = _layer_body_halves
        out_specs = pl.BlockSpec((2, bn, dout // 2), lambda i: (0, i, 0))
        out_shape = jax.ShapeDtypeStruct((2, n, dout // 2), jnp.float32)
    else:
        body = _layer_body_single
        out_specs = pl.BlockSpec((bn, dout), lambda i: (i, 0))
        out_shape = jax.ShapeDtypeStruct((n, dout), jnp.float32)
    return pl.pallas_call(
        body,
        grid=(n // bn,),
        in_specs=in_specs,
        out_specs=out_specs,
        out_shape=out_shape,
        interpret=interpret,
    )


def _t3_body(p_ref, g2_ref, dinv_ref, w_ref, b_ref, lg_ref, gn_ref,
             out_ref, *, temp):
    i = pl.program_id(0)
    dinv = dinv_ref[...]
    agg_a = (p_ref[0] + g2_ref[0]) * dinv
    agg_b = (p_ref[1] + g2_ref[1]) * dinv
    half = w_ref.shape[0] // 2
    h = (jnp.dot(agg_a, w_ref[:half], preferred_element_type=jnp.float32)
         + jnp.dot(agg_b, w_ref[half:], preferred_element_type=jnp.float32))
    h = jnp.maximum(h + b_ref[...], 0.0)
    t = (lg_ref[...] + gn_ref[...]) * (1.0 / temp)
    t = t - jnp.max(t, axis=1, keepdims=True)
    e = jnp.exp(t)
    y = e / jnp.sum(e, axis=1, keepdims=True)
    contrib = lax.dot_general(y, h, (((0,), (0,)), ((), ())),
                              preferred_element_type=jnp.float32)

    @pl.when(i == 0)
    def _():
        out_ref[...] = contrib

    @pl.when(i > 0)
    def _():
        out_ref[...] += contrib


@functools.lru_cache(None)
def _t3(n, n_pad, bn, k, dout, temp, interpret=False):
    return pl.pallas_call(
        functools.partial(_t3_body, temp=temp),
        grid=(n // bn,),
        in_specs=[pl.BlockSpec((2, bn, 16), lambda i: (0, i, 0)),
                  pl.BlockSpec((2, bn, 16), lambda i: (0, i, 0)),
                  pl.BlockSpec((bn, 1), lambda i: (i, 0)),
                  pl.BlockSpec((32, dout), lambda i: (0, 0)),
                  pl.BlockSpec((1, dout), lambda i: (0, 0)),
                  pl.BlockSpec((bn, k), lambda i: (i, 0)),
                  pl.BlockSpec((bn, k), lambda i: (i, 0))],
        out_specs=pl.BlockSpec((k, dout), lambda i: (0, 0)),
        out_shape=jax.ShapeDtypeStruct((k, dout), jnp.float32),
        interpret=interpret,
    )


def kernel(x, edge_index, logits, gumbel_noise, W1, b1, W2, b2, W3, b3):
    n = x.shape[0]
    e = edge_index.shape[1]
    k = logits.shape[1]
    bn = 2000
    temp = 0.5

    e_pad = _ceil_to(e, _NW * _R * _CH)
    n_pad = _ceil_to(n + 1, _NS * 8 * 2)
    trash = n

    src = edge_index[0]
    dst = edge_index[1]
    pad_e = e_pad - e
    src_p = jnp.concatenate([src, jnp.zeros((pad_e,), jnp.int32)])
    dst_p = jnp.concatenate([dst, jnp.full((pad_e,), trash, jnp.int32)])
    rows_idx = e_pad // _CH
    src2d = src_p.reshape(rows_idx, _CH)
    dst2d = dst_p.reshape(rows_idx, _CH)
    ei2 = jnp.stack([src2d, dst2d], axis=1)
    ei3 = jnp.stack([jnp.stack([src2d, dst2d], axis=1),
                     jnp.stack([src2d + n, dst2d], axis=1)])
    rpt_split = rows_idx // _NW
    rpt_all = rows_idx // _NS

    slice_rows = n_pad // _NS
    zeros8 = jnp.zeros((slice_rows, 8), jnp.float32)
    zeros16 = jnp.zeros((slice_rows, 16), jnp.float32)
    ones128 = jnp.ones((_CH, 8), jnp.float32)

    degp = _sc_degree(n_pad, rpt_split)(dst2d, ones128, zeros8)
    dinv, g0 = _t0(n, n_pad, bn)(degp, x)
    p1 = _sc_agg(8, n_pad, rpt_split, True)(ei2, g0, zeros8)
    w1p = jnp.concatenate([W1, jnp.zeros((6, W1.shape[1]), W1.dtype)], axis=0)
    g1 = _t_layer(n, n_pad, bn, 8, 16, False)(
        p1, g0, dinv, w1p, b1.reshape(1, -1))
    p2 = _sc_agg(16, n_pad, rpt_split, True, 4)(ei2, g1, zeros16)
    g2 = _t_layer(n, n_pad, bn, 16, 32, True)(
        p2, g1, dinv, W2, b2.reshape(1, -1))
    g2flat = g2.reshape(2 * n, 16)
    p3 = _sc_agg(16, n_pad, rpt_all, False, 4)(ei3, g2flat, zeros16)
    out = _t3(n, n_pad, bn, k, W3.shape[1], temp)(
        p3, g2, dinv, W3, b3.reshape(1, -1),
        logits, gumbel_noise)
    return out.reshape(1, -1)
